# baseline (speedup 1.0000x reference)
import functools

import jax
import jax.numpy as jnp
from jax import lax
from jax.experimental import pallas as pl
from jax.experimental.pallas import tpu as pltpu
from jax.experimental.pallas import tpu_sc as plsc

N_NODES = 50000
C_IN = 128
C_OUT = 128
CW = C_OUT // 2
K = 9
NW = 32
B = 96
NCHUNK = 17
NPAD = NW * B * NCHUNK
ROWS_PER_W = NPAD // NW
MM_BLK = 1024


def _rne16(u):
    lsb = lax.shift_right_logical(u, 16) & 1
    return lax.shift_right_logical(u + 32767 + lsb, 16)


def _pack(cols):
    ua = _rne16(lax.bitcast_convert_type(cols[:, :CW], jnp.int32))
    ub = _rne16(lax.bitcast_convert_type(cols[:, CW:], jnp.int32))
    return ua | (ub << 16)


def _mm_body(x_ref, w_ref, y_ref):
    yf = jnp.dot(x_ref[...], w_ref[...], preferred_element_type=jnp.float32)
    for k in range(K):
        ev = _pack(yf[:, k * C_IN: (k + 1) * C_IN])
        od = _pack(yf[:, K * C_IN + k * C_IN: K * C_IN + (k + 1) * C_IN])
        y_ref[k] = jnp.concatenate([ev, od], axis=1)


def _tc_matmul(fpad2, w2d):
    return pl.pallas_call(
        _mm_body,
        grid=(NPAD // MM_BLK,),
        in_specs=[
            pl.BlockSpec((MM_BLK // 2, 2 * C_IN), lambda i: (i, 0)),
            pl.BlockSpec((2 * C_IN, 2 * K * C_OUT), lambda i: (0, 0)),
        ],
        out_specs=pl.BlockSpec((K, MM_BLK // 2, 2 * CW), lambda i: (0, i, 0)),
        out_shape=jax.ShapeDtypeStruct((K, NPAD // 2, 2 * CW), jnp.int32),
    )(fpad2, w2d)


def _remap_body(idx_ref, gidx_ref):
    v = idx_ref[...]
    koff = lax.broadcasted_iota(jnp.int32, (K, NPAD), 0) * NPAD
    gidx_ref[...] = jnp.where(v < 0, N_NODES, v) + koff


def _tc_remap(idx_t):
    return pl.pallas_call(
        _remap_body,
        out_shape=jax.ShapeDtypeStruct((K, NPAD), jnp.int32),
    )(idx_t)


def _sc_body(y_hbm, gidx_hbm, b_hbm, out_hbm,
             b_v, gidx_v, rows0, out0, gsem0, gsem1, gsem2):
    wid = lax.axis_index("s") * 2 + lax.axis_index("c")
    pltpu.sync_copy(b_hbm, b_v)
    pltpu.sync_copy(gidx_hbm.at[wid], gidx_v)
    wbase = wid * ROWS_PER_W
    himask = jnp.int32(-65536)

    gsems = (gsem0, gsem1, gsem2)

    def chunk(c, carry):
        copies = [pltpu.async_copy(y_hbm.at[gidx_v.at[k, c]], rows0.at[k],
                                   gsems[k // 3]) for k in range(K)]
        for g in range(3):
            for k in range(3 * g, 3 * g + 3):
                copies[k].wait()

            def acc(i, carry2, g=g):
                for seg in range(4):
                    s = pl.ds(seg * 16, 16)
                    hs = pl.ds(64 + seg * 16, 16)
                    if g == 0:
                        alo = b_v[s]
                        ahi = b_v[hs]
                    else:
                        alo = out0[i, s]
                        ahi = out0[i, hs]
                    for k in range(3 * g, 3 * g + 3):
                        v = rows0[k, i, s]
                        alo = alo + lax.bitcast_convert_type(v << 16,
                                                             jnp.float32)
                        ahi = ahi + lax.bitcast_convert_type(v & himask,
                                                             jnp.float32)
                    out0[i, s] = alo
                    out0[i, hs] = ahi
                return carry2

            lax.fori_loop(0, B, acc, 0)
        pltpu.sync_copy(out0, out_hbm.at[pl.ds(wbase + c * B, B)])
        return carry

    lax.fori_loop(0, NCHUNK, chunk, 0)


def _sc_gather_sum(yflat, gidx4, b):
    mesh = plsc.VectorSubcoreMesh(core_axis_name="c", subcore_axis_name="s")
    fn = pl.kernel(
        _sc_body,
        mesh=mesh,
        out_type=jax.ShapeDtypeStruct((NPAD, C_OUT), jnp.float32),
        compiler_params=pltpu.CompilerParams(use_tc_tiling_on_sc=False),
        scratch_types=[
            pltpu.VMEM((C_OUT,), jnp.float32),
            pltpu.VMEM((K, NCHUNK, B), jnp.int32),
            pltpu.VMEM((K, B, CW), jnp.int32),
            pltpu.VMEM((B, C_OUT), jnp.float32),
            pltpu.SemaphoreType.DMA,
            pltpu.SemaphoreType.DMA,
            pltpu.SemaphoreType.DMA,
        ],
    )
    return fn(yflat, gidx4, b)


def kernel(features, neigh_idx, W, b):
    N, C = features.shape
    fpad2 = jnp.zeros((NPAD // 2, 2 * C), jnp.bfloat16).at[:N // 2].set(
        features.astype(jnp.bfloat16).reshape(N // 2, 2 * C))
    w2 = W.reshape(C_OUT, K, C_IN).transpose(2, 1, 0).reshape(C_IN, K * C_OUT)
    w2d = jnp.zeros((2 * C_IN, 2 * K * C_OUT), w2.dtype)
    w2d = w2d.at[:C_IN, :K * C_OUT].set(w2)
    w2d = w2d.at[C_IN:, K * C_OUT:].set(w2)
    y = _tc_matmul(fpad2, w2d.astype(jnp.bfloat16))
    yflat = y.reshape(K * NPAD, CW)
    idx_pad = jnp.full((NPAD, K), -1, neigh_idx.dtype).at[:N].set(neigh_idx)
    idx_t = idx_pad.T.astype(jnp.int32)
    gidx = _tc_remap(idx_t)
    gidx4 = gidx.reshape(K, NW, NCHUNK, B).transpose(1, 0, 2, 3)
    out = _sc_gather_sum(yflat, gidx4, b)
    return out[:N]


# speedup vs baseline: 1.2594x; 1.2594x over previous
import functools

import jax
import jax.numpy as jnp
from jax import lax
from jax.experimental import pallas as pl
from jax.experimental.pallas import tpu as pltpu
from jax.experimental.pallas import tpu_sc as plsc

N_NODES = 50000
C_IN = 128
C_OUT = 128
CW = C_OUT // 2
K = 9
NW = 32
B = 64
NCHUNK = 25
NPAD = NW * B * NCHUNK
ROWS_PER_W = NPAD // NW
MM_BLK = 1024


def _rne16(u):
    lsb = lax.shift_right_logical(u, 16) & 1
    return lax.shift_right_logical(u + 32767 + lsb, 16)


def _pack(cols):
    ua = _rne16(lax.bitcast_convert_type(cols[:, :CW], jnp.int32))
    ub = _rne16(lax.bitcast_convert_type(cols[:, CW:], jnp.int32))
    return ua | (ub << 16)


def _mm_body(x_ref, w_ref, y_ref):
    yf = jnp.dot(x_ref[...], w_ref[...], preferred_element_type=jnp.float32)
    for k in range(K):
        ev = _pack(yf[:, k * C_IN: (k + 1) * C_IN])
        od = _pack(yf[:, K * C_IN + k * C_IN: K * C_IN + (k + 1) * C_IN])
        y_ref[k] = jnp.concatenate([ev, od], axis=1)


def _tc_matmul(fpad2, w2d):
    return pl.pallas_call(
        _mm_body,
        grid=(NPAD // MM_BLK,),
        in_specs=[
            pl.BlockSpec((MM_BLK // 2, 2 * C_IN), lambda i: (i, 0)),
            pl.BlockSpec((2 * C_IN, 2 * K * C_OUT), lambda i: (0, 0)),
        ],
        out_specs=pl.BlockSpec((K, MM_BLK // 2, 2 * CW), lambda i: (0, i, 0)),
        out_shape=jax.ShapeDtypeStruct((K, NPAD // 2, 2 * CW), jnp.int32),
    )(fpad2, w2d)


def _remap_body(idx_ref, gidx_ref):
    v = idx_ref[...]
    koff = lax.broadcasted_iota(jnp.int32, (K, NPAD), 0) * NPAD
    gidx_ref[...] = jnp.where(v < 0, N_NODES, v) + koff


def _tc_remap(idx_t):
    return pl.pallas_call(
        _remap_body,
        out_shape=jax.ShapeDtypeStruct((K, NPAD), jnp.int32),
    )(idx_t)


def _sc_body(y_hbm, gidx_hbm, b_hbm, out_hbm,
             b_v, gidx_v, rows0, rows1, out0, gsem0, gsem1):
    wid = lax.axis_index("s") * 2 + lax.axis_index("c")
    pltpu.sync_copy(b_hbm, b_v)
    pltpu.sync_copy(gidx_hbm.at[wid], gidx_v)
    wbase = wid * ROWS_PER_W
    himask = jnp.int32(-65536)

    rows = (rows0, rows1)
    gsems = (gsem0, gsem1)

    def fire(c, par):
        return [pltpu.async_copy(y_hbm.at[gidx_v.at[k, c]], rows[par].at[k],
                                 gsems[par]) for k in range(K)]

    pending = fire(0, 0)
    for c in range(NCHUNK):
        par = c % 2
        nxt = fire(c + 1, 1 - par) if c + 1 < NCHUNK else []
        for cp in pending:
            cp.wait()
        pending = nxt
        buf = rows[par]

        def acc(i, carry2, buf=buf):
            for seg in range(4):
                s = pl.ds(seg * 16, 16)
                alo = b_v[s]
                ahi = b_v[pl.ds(64 + seg * 16, 16)]
                for k in range(K):
                    v = buf[k, i, s]
                    alo = alo + lax.bitcast_convert_type(v << 16, jnp.float32)
                    ahi = ahi + lax.bitcast_convert_type(v & himask,
                                                         jnp.float32)
                out0[i, s] = alo
                out0[i, pl.ds(64 + seg * 16, 16)] = ahi
            return carry2

        lax.fori_loop(0, B, acc, 0)
        pltpu.sync_copy(out0, out_hbm.at[pl.ds(wbase + c * B, B)])


def _sc_gather_sum(yflat, gidx4, b):
    mesh = plsc.VectorSubcoreMesh(core_axis_name="c", subcore_axis_name="s")
    fn = pl.kernel(
        _sc_body,
        mesh=mesh,
        out_type=jax.ShapeDtypeStruct((NPAD, C_OUT), jnp.float32),
        compiler_params=pltpu.CompilerParams(use_tc_tiling_on_sc=False),
        scratch_types=[
            pltpu.VMEM((C_OUT,), jnp.float32),
            pltpu.VMEM((K, NCHUNK, B), jnp.int32),
            pltpu.VMEM((K, B, CW), jnp.int32),
            pltpu.VMEM((K, B, CW), jnp.int32),
            pltpu.VMEM((B, C_OUT), jnp.float32),
            pltpu.SemaphoreType.DMA,
            pltpu.SemaphoreType.DMA,
        ],
    )
    return fn(yflat, gidx4, b)


def kernel(features, neigh_idx, W, b):
    N, C = features.shape
    fpad2 = jnp.zeros((NPAD // 2, 2 * C), jnp.bfloat16).at[:N // 2].set(
        features.astype(jnp.bfloat16).reshape(N // 2, 2 * C))
    w2 = W.reshape(C_OUT, K, C_IN).transpose(2, 1, 0).reshape(C_IN, K * C_OUT)
    w2d = jnp.zeros((2 * C_IN, 2 * K * C_OUT), w2.dtype)
    w2d = w2d.at[:C_IN, :K * C_OUT].set(w2)
    w2d = w2d.at[C_IN:, K * C_OUT:].set(w2)
    y = _tc_matmul(fpad2, w2d.astype(jnp.bfloat16))
    yflat = y.reshape(K * NPAD, CW)
    idx_pad = jnp.full((NPAD, K), -1, neigh_idx.dtype).at[:N].set(neigh_idx)
    idx_t = idx_pad.T.astype(jnp.int32)
    gidx = _tc_remap(idx_t)
    gidx4 = gidx.reshape(K, NW, NCHUNK, B).transpose(1, 0, 2, 3)
    out = _sc_gather_sum(yflat, gidx4, b)
    return out[:N]


# speedup vs baseline: 1.2986x; 1.0311x over previous
import functools

import jax
import jax.numpy as jnp
from jax import lax
from jax.experimental import pallas as pl
from jax.experimental.pallas import tpu as pltpu
from jax.experimental.pallas import tpu_sc as plsc

N_NODES = 50000
C_IN = 128
C_OUT = 128
CW = C_OUT // 2
K = 9
NW = 32
B = 64
NCHUNK = 25
NPAD = NW * B * NCHUNK
ROWS_PER_W = NPAD // NW
MM_BLK = 2048


def _rne16(u):
    lsb = lax.shift_right_logical(u, 16) & 1
    return lax.shift_right_logical(u + 32767 + lsb, 16)


def _pack(cols):
    ua = _rne16(lax.bitcast_convert_type(cols[:, :CW], jnp.int32))
    ub = _rne16(lax.bitcast_convert_type(cols[:, CW:], jnp.int32))
    return ua | (ub << 16)


def _mm_body(x_ref, w_ref, y_ref):
    yf = jnp.dot(x_ref[...], w_ref[...], preferred_element_type=jnp.float32)
    for k in range(K):
        ev = _pack(yf[:, k * C_IN: (k + 1) * C_IN])
        od = _pack(yf[:, K * C_IN + k * C_IN: K * C_IN + (k + 1) * C_IN])
        y_ref[k] = jnp.concatenate([ev, od], axis=1)


def _tc_matmul(fpad2, w2d):
    return pl.pallas_call(
        _mm_body,
        grid=(NPAD // MM_BLK,),
        in_specs=[
            pl.BlockSpec((MM_BLK // 2, 2 * C_IN), lambda i: (i, 0)),
            pl.BlockSpec((2 * C_IN, 2 * K * C_OUT), lambda i: (0, 0)),
        ],
        out_specs=pl.BlockSpec((K, MM_BLK // 2, 2 * CW), lambda i: (0, i, 0)),
        out_shape=jax.ShapeDtypeStruct((K, NPAD // 2, 2 * CW), jnp.int32),
    )(fpad2, w2d)


def _remap_body(idx_ref, gidx_ref):
    v = idx_ref[...]
    koff = lax.broadcasted_iota(jnp.int32, (K, NPAD), 0) * NPAD
    gidx_ref[...] = jnp.where(v < 0, N_NODES, v) + koff


def _tc_remap(idx_t):
    return pl.pallas_call(
        _remap_body,
        out_shape=jax.ShapeDtypeStruct((K, NPAD), jnp.int32),
    )(idx_t)


def _sc_body(y_hbm, gidx_hbm, b_hbm, out_hbm,
             b_v, gidx_v, rows0, rows1, out0, gsem0, gsem1):
    wid = lax.axis_index("s") * 2 + lax.axis_index("c")
    pltpu.sync_copy(b_hbm, b_v)
    pltpu.sync_copy(gidx_hbm.at[wid], gidx_v)
    wbase = wid * ROWS_PER_W
    himask = jnp.int32(-65536)

    rows = (rows0, rows1)
    gsems = (gsem0, gsem1)

    def fire(c, par):
        return [pltpu.async_copy(y_hbm.at[gidx_v.at[k, c]], rows[par].at[k],
                                 gsems[par]) for k in range(K)]

    pending = fire(0, 0)
    for c in range(NCHUNK):
        par = c % 2
        nxt = fire(c + 1, 1 - par) if c + 1 < NCHUNK else []
        for cp in pending:
            cp.wait()
        pending = nxt
        buf = rows[par]

        def acc(i, carry2, buf=buf):
            for seg in range(4):
                s = pl.ds(seg * 16, 16)
                alo = b_v[s]
                ahi = b_v[pl.ds(64 + seg * 16, 16)]
                for k in range(K):
                    v = buf[k, i, s]
                    alo = alo + lax.bitcast_convert_type(v << 16, jnp.float32)
                    ahi = ahi + lax.bitcast_convert_type(v & himask,
                                                         jnp.float32)
                out0[i, s] = alo
                out0[i, pl.ds(64 + seg * 16, 16)] = ahi
            return carry2

        lax.fori_loop(0, B, acc, 0)
        pltpu.sync_copy(out0, out_hbm.at[pl.ds(wbase + c * B, B)])


def _sc_gather_sum(yflat, gidx4, b):
    mesh = plsc.VectorSubcoreMesh(core_axis_name="c", subcore_axis_name="s")
    fn = pl.kernel(
        _sc_body,
        mesh=mesh,
        out_type=jax.ShapeDtypeStruct((NPAD, C_OUT), jnp.float32),
        compiler_params=pltpu.CompilerParams(use_tc_tiling_on_sc=False),
        scratch_types=[
            pltpu.VMEM((C_OUT,), jnp.float32),
            pltpu.VMEM((K, NCHUNK, B), jnp.int32),
            pltpu.VMEM((K, B, CW), jnp.int32),
            pltpu.VMEM((K, B, CW), jnp.int32),
            pltpu.VMEM((B, C_OUT), jnp.float32),
            pltpu.SemaphoreType.DMA,
            pltpu.SemaphoreType.DMA,
        ],
    )
    return fn(yflat, gidx4, b)


def kernel(features, neigh_idx, W, b):
    N, C = features.shape
    fpad2 = jnp.zeros((NPAD // 2, 2 * C), jnp.bfloat16).at[:N // 2].set(
        features.astype(jnp.bfloat16).reshape(N // 2, 2 * C))
    w2 = W.reshape(C_OUT, K, C_IN).transpose(2, 1, 0).reshape(C_IN, K * C_OUT)
    w2d = jnp.zeros((2 * C_IN, 2 * K * C_OUT), w2.dtype)
    w2d = w2d.at[:C_IN, :K * C_OUT].set(w2)
    w2d = w2d.at[C_IN:, K * C_OUT:].set(w2)
    y = _tc_matmul(fpad2, w2d.astype(jnp.bfloat16))
    yflat = y.reshape(K * NPAD, CW)
    idx_pad = jnp.full((NPAD, K), -1, neigh_idx.dtype).at[:N].set(neigh_idx)
    idx_t = idx_pad.T.astype(jnp.int32)
    gidx = _tc_remap(idx_t)
    gidx4 = gidx.reshape(K, NW, NCHUNK, B).transpose(1, 0, 2, 3)
    out = _sc_gather_sum(yflat, gidx4, b)
    return out[:N]


# speedup vs baseline: 1.3038x; 1.0041x over previous
import functools

import jax
import jax.numpy as jnp
from jax import lax
from jax.experimental import pallas as pl
from jax.experimental.pallas import tpu as pltpu
from jax.experimental.pallas import tpu_sc as plsc

N_NODES = 50000
C_IN = 128
C_OUT = 128
CW = C_OUT // 2
K = 9
NW = 32
B = 64
NCHUNK = 25
NPAD = NW * B * NCHUNK
ROWS_PER_W = NPAD // NW
MM_BLK = 2048


def _rne16(u):
    lsb = lax.shift_right_logical(u, 16) & 1
    return lax.shift_right_logical(u + 32767 + lsb, 16)


def _pack(cols):
    ua = _rne16(lax.bitcast_convert_type(cols[:, :CW], jnp.int32))
    ub = _rne16(lax.bitcast_convert_type(cols[:, CW:], jnp.int32))
    return ua | (ub << 16)


def _mm_body(x_ref, w_ref, y_ref):
    yf = jnp.dot(x_ref[...], w_ref[...], preferred_element_type=jnp.float32)
    for k in range(K):
        ev = _pack(yf[:, k * C_IN: (k + 1) * C_IN])
        od = _pack(yf[:, K * C_IN + k * C_IN: K * C_IN + (k + 1) * C_IN])
        y_ref[k] = jnp.concatenate([ev, od], axis=1)


def _tc_matmul(fpad2, w2d):
    return pl.pallas_call(
        _mm_body,
        grid=(NPAD // MM_BLK,),
        in_specs=[
            pl.BlockSpec((MM_BLK // 2, 2 * C_IN), lambda i: (i, 0)),
            pl.BlockSpec((2 * C_IN, 2 * K * C_OUT), lambda i: (0, 0)),
        ],
        out_specs=pl.BlockSpec((K, MM_BLK // 2, 2 * CW), lambda i: (0, i, 0)),
        out_shape=jax.ShapeDtypeStruct((K, NPAD // 2, 2 * CW), jnp.int32),
    )(fpad2, w2d)


def _remap_body(idx_ref, gidx_ref):
    v = idx_ref[...]
    koff = lax.broadcasted_iota(jnp.int32, (K, NPAD), 0) * NPAD
    gidx_ref[...] = jnp.where(v < 0, N_NODES, v) + koff


def _tc_remap(idx_t):
    return pl.pallas_call(
        _remap_body,
        out_shape=jax.ShapeDtypeStruct((K, NPAD), jnp.int32),
    )(idx_t)


def _sc_body(y_hbm, gidx_hbm, b_hbm, out_hbm,
             b_v, gidx_v, rows0, rows1, out0, out1,
             gsem0, gsem1, osem0, osem1):
    wid = lax.axis_index("s") * 2 + lax.axis_index("c")
    pltpu.sync_copy(b_hbm, b_v)
    pltpu.sync_copy(gidx_hbm.at[wid], gidx_v)
    wbase = wid * ROWS_PER_W
    himask = jnp.int32(-65536)

    rows = (rows0, rows1)
    gsems = (gsem0, gsem1)
    outs = (out0, out1)
    osems = (osem0, osem1)
    ostores = [None, None]

    def fire(c, par):
        return [pltpu.async_copy(y_hbm.at[gidx_v.at[k, c]], rows[par].at[k],
                                 gsems[par]) for k in range(K)]

    pending = fire(0, 0)
    for c in range(NCHUNK):
        par = c % 2
        nxt = fire(c + 1, 1 - par) if c + 1 < NCHUNK else []
        for cp in pending:
            cp.wait()
        pending = nxt
        buf = rows[par]
        outb = outs[par]
        if ostores[par] is not None:
            ostores[par].wait()

        def acc(i, carry2, buf=buf, outb=outb):
            for seg in range(4):
                s = pl.ds(seg * 16, 16)
                alo = b_v[s]
                ahi = b_v[pl.ds(64 + seg * 16, 16)]
                for k in range(K):
                    v = buf[k, i, s]
                    alo = alo + lax.bitcast_convert_type(v << 16, jnp.float32)
                    ahi = ahi + lax.bitcast_convert_type(v & himask,
                                                         jnp.float32)
                outb[i, s] = alo
                outb[i, pl.ds(64 + seg * 16, 16)] = ahi
            return carry2

        lax.fori_loop(0, B, acc, 0)
        ostores[par] = pltpu.async_copy(
            outb, out_hbm.at[pl.ds(wbase + c * B, B)], osems[par])
    ostores[0].wait()
    ostores[1].wait()


def _sc_gather_sum(yflat, gidx4, b):
    mesh = plsc.VectorSubcoreMesh(core_axis_name="c", subcore_axis_name="s")
    fn = pl.kernel(
        _sc_body,
        mesh=mesh,
        out_type=jax.ShapeDtypeStruct((NPAD, C_OUT), jnp.float32),
        compiler_params=pltpu.CompilerParams(use_tc_tiling_on_sc=False),
        scratch_types=[
            pltpu.VMEM((C_OUT,), jnp.float32),
            pltpu.VMEM((K, NCHUNK, B), jnp.int32),
            pltpu.VMEM((K, B, CW), jnp.int32),
            pltpu.VMEM((K, B, CW), jnp.int32),
            pltpu.VMEM((B, C_OUT), jnp.float32),
            pltpu.VMEM((B, C_OUT), jnp.float32),
            pltpu.SemaphoreType.DMA,
            pltpu.SemaphoreType.DMA,
            pltpu.SemaphoreType.DMA,
            pltpu.SemaphoreType.DMA,
        ],
    )
    return fn(yflat, gidx4, b)


def kernel(features, neigh_idx, W, b):
    N, C = features.shape
    fpad2 = jnp.zeros((NPAD // 2, 2 * C), jnp.bfloat16).at[:N // 2].set(
        features.astype(jnp.bfloat16).reshape(N // 2, 2 * C))
    w2 = W.reshape(C_OUT, K, C_IN).transpose(2, 1, 0).reshape(C_IN, K * C_OUT)
    w2d = jnp.zeros((2 * C_IN, 2 * K * C_OUT), w2.dtype)
    w2d = w2d.at[:C_IN, :K * C_OUT].set(w2)
    w2d = w2d.at[C_IN:, K * C_OUT:].set(w2)
    y = _tc_matmul(fpad2, w2d.astype(jnp.bfloat16))
    yflat = y.reshape(K * NPAD, CW)
    idx_pad = jnp.full((NPAD, K), -1, neigh_idx.dtype).at[:N].set(neigh_idx)
    idx_t = idx_pad.T.astype(jnp.int32)
    gidx = _tc_remap(idx_t)
    gidx4 = gidx.reshape(K, NW, NCHUNK, B).transpose(1, 0, 2, 3)
    out = _sc_gather_sum(yflat, gidx4, b)
    return out[:N]


# speedup vs baseline: 1.4360x; 1.1014x over previous
import functools

import jax
import jax.numpy as jnp
from jax import lax
from jax.experimental import pallas as pl
from jax.experimental.pallas import tpu as pltpu
from jax.experimental.pallas import tpu_sc as plsc

N_NODES = 50000
C_IN = 128
C_OUT = 128
CW = C_OUT // 2
K = 9
NW = 32
B = 64
NCHUNK = 25
NPAD = NW * B * NCHUNK
ROWS_PER_W = NPAD // NW
MM_BLK = 2048


def _rne16(u):
    lsb = lax.shift_right_logical(u, 16) & 1
    return lax.shift_right_logical(u + 32767 + lsb, 16)


def _pack(cols):
    ua = _rne16(lax.bitcast_convert_type(cols[:, :CW], jnp.int32))
    ub = _rne16(lax.bitcast_convert_type(cols[:, CW:], jnp.int32))
    return ua | (ub << 16)


def _mm_body(x1_ref, x2_ref, w_ref, y_ref):
    y1 = jnp.dot(x1_ref[...], w_ref[...], preferred_element_type=jnp.float32)
    y2 = jnp.dot(x2_ref[...], w_ref[...], preferred_element_type=jnp.float32)
    for k in range(K):
        lo = _pack(y1[:, k * C_IN: (k + 1) * C_IN])
        hi = _pack(y2[:, k * C_IN: (k + 1) * C_IN])
        y_ref[k] = jnp.concatenate([lo, hi], axis=1)


BLK2 = MM_BLK // 2
H = None


def _tc_matmul(fpad, w2):
    nblk = NPAD // 2 // BLK2
    return pl.pallas_call(
        _mm_body,
        grid=(nblk,),
        in_specs=[
            pl.BlockSpec((BLK2, C_IN), lambda i: (i, 0)),
            pl.BlockSpec((BLK2, C_IN), lambda i, nblk=nblk: (i + nblk, 0)),
            pl.BlockSpec((C_IN, K * C_OUT), lambda i: (0, 0)),
        ],
        out_specs=pl.BlockSpec((K, BLK2, 2 * CW), lambda i: (0, i, 0)),
        out_shape=jax.ShapeDtypeStruct((K, NPAD // 2, 2 * CW), jnp.int32),
    )(fpad, fpad, w2)


def _remap_body(idx_ref, gidx_ref):
    v = idx_ref[...]
    h = NPAD // 2
    kk = lax.broadcasted_iota(jnp.int32, (K, NPAD), 0)
    n = jnp.where(v < 0, N_NODES, v)
    gidx_ref[...] = jnp.where(n < h,
                              2 * (kk * h + n),
                              2 * (kk * h + n - h) + 1)


def _tc_remap(idx_t):
    return pl.pallas_call(
        _remap_body,
        out_shape=jax.ShapeDtypeStruct((K, NPAD), jnp.int32),
    )(idx_t)


def _sc_body(y_hbm, gidx_hbm, b_hbm, out_hbm,
             b_v, gidx_v, rows0, rows1, out0, out1,
             gsem0, gsem1, osem0, osem1):
    wid = lax.axis_index("s") * 2 + lax.axis_index("c")
    pltpu.sync_copy(b_hbm, b_v)
    pltpu.sync_copy(gidx_hbm.at[wid], gidx_v)
    wbase = wid * ROWS_PER_W
    himask = jnp.int32(-65536)

    rows = (rows0, rows1)
    gsems = (gsem0, gsem1)
    outs = (out0, out1)
    osems = (osem0, osem1)
    ostores = [None, None]

    def fire(c, par):
        return [pltpu.async_copy(y_hbm.at[gidx_v.at[k, c]], rows[par].at[k],
                                 gsems[par]) for k in range(K)]

    pending = fire(0, 0)
    for c in range(NCHUNK):
        par = c % 2
        nxt = fire(c + 1, 1 - par) if c + 1 < NCHUNK else []
        for cp in pending:
            cp.wait()
        pending = nxt
        buf = rows[par]
        outb = outs[par]
        if ostores[par] is not None:
            ostores[par].wait()

        def acc(i, carry2, buf=buf, outb=outb):
            for seg in range(4):
                s = pl.ds(seg * 16, 16)
                alo = b_v[s]
                ahi = b_v[pl.ds(64 + seg * 16, 16)]
                for k in range(K):
                    v = buf[k, i, s]
                    alo = alo + lax.bitcast_convert_type(v << 16, jnp.float32)
                    ahi = ahi + lax.bitcast_convert_type(v & himask,
                                                         jnp.float32)
                outb[i, s] = alo
                outb[i, pl.ds(64 + seg * 16, 16)] = ahi
            return carry2

        lax.fori_loop(0, B, acc, 0)
        ostores[par] = pltpu.async_copy(
            outb, out_hbm.at[pl.ds(wbase + c * B, B)], osems[par])
    ostores[0].wait()
    ostores[1].wait()


def _sc_gather_sum(yflat, gidx4, b):
    mesh = plsc.VectorSubcoreMesh(core_axis_name="c", subcore_axis_name="s")
    fn = pl.kernel(
        _sc_body,
        mesh=mesh,
        out_type=jax.ShapeDtypeStruct((NPAD, C_OUT), jnp.float32),
        compiler_params=pltpu.CompilerParams(use_tc_tiling_on_sc=False),
        scratch_types=[
            pltpu.VMEM((C_OUT,), jnp.float32),
            pltpu.VMEM((K, NCHUNK, B), jnp.int32),
            pltpu.VMEM((K, B, CW), jnp.int32),
            pltpu.VMEM((K, B, CW), jnp.int32),
            pltpu.VMEM((B, C_OUT), jnp.float32),
            pltpu.VMEM((B, C_OUT), jnp.float32),
            pltpu.SemaphoreType.DMA,
            pltpu.SemaphoreType.DMA,
            pltpu.SemaphoreType.DMA,
            pltpu.SemaphoreType.DMA,
        ],
    )
    return fn(yflat, gidx4, b)


def kernel(features, neigh_idx, W, b):
    N, C = features.shape
    fpad = jnp.zeros((NPAD, C), jnp.bfloat16).at[:N].set(
        features.astype(jnp.bfloat16))
    w2 = W.reshape(C_OUT, K, C_IN).transpose(2, 1, 0).reshape(C_IN, K * C_OUT)
    y = _tc_matmul(fpad, w2.astype(jnp.bfloat16))
    yflat = y.reshape(K * NPAD, CW)
    idx_pad = jnp.full((NPAD, K), -1, neigh_idx.dtype).at[:N].set(neigh_idx)
    idx_t = idx_pad.T.astype(jnp.int32)
    gidx = _tc_remap(idx_t)
    gidx4 = gidx.reshape(K, NW, NCHUNK, B).transpose(1, 0, 2, 3)
    out = _sc_gather_sum(yflat, gidx4, b)
    return out[:N]


# speedup vs baseline: 1.4387x; 1.0019x over previous
import jax
import jax.numpy as jnp
from jax import lax
from jax.experimental import pallas as pl
from jax.experimental.pallas import tpu as pltpu
from jax.experimental.pallas import tpu_sc as plsc

N_NODES = 50000
C_IN = 128
C_OUT = 128
CW = C_OUT // 2
K = 9
NW = 32
B = 64
NCHUNK = 25
NPAD = NW * B * NCHUNK
ROWS_PER_W = NPAD // NW
MM_BLK = 2048


def _rne16(u):
    lsb = lax.shift_right_logical(u, 16) & 1
    return lax.shift_right_logical(u + 32767 + lsb, 16)


def _pack(cols):
    ua = _rne16(lax.bitcast_convert_type(cols[:, :CW], jnp.int32))
    ub = _rne16(lax.bitcast_convert_type(cols[:, CW:], jnp.int32))
    return ua | (ub << 16)


def _mm_body(x1_ref, x2_ref, w_ref, y_ref):
    y1 = jnp.dot(x1_ref[...], w_ref[...], preferred_element_type=jnp.float32)
    y2 = jnp.dot(x2_ref[...], w_ref[...], preferred_element_type=jnp.float32)
    for k in range(K):
        lo = _pack(y1[:, k * C_IN: (k + 1) * C_IN])
        hi = _pack(y2[:, k * C_IN: (k + 1) * C_IN])
        y_ref[k] = jnp.concatenate([lo, hi], axis=1)


BLK2 = MM_BLK // 2


def _tc_matmul(fpad, w2):
    nblk = NPAD // 2 // BLK2
    return pl.pallas_call(
        _mm_body,
        grid=(nblk,),
        in_specs=[
            pl.BlockSpec((BLK2, C_IN), lambda i: (i, 0)),
            pl.BlockSpec((BLK2, C_IN), lambda i, nblk=nblk: (i + nblk, 0)),
            pl.BlockSpec((C_IN, K * C_OUT), lambda i: (0, 0)),
        ],
        out_specs=pl.BlockSpec((K, BLK2, 2 * CW), lambda i: (0, i, 0)),
        out_shape=jax.ShapeDtypeStruct((K, NPAD // 2, 2 * CW), jnp.int32),
    )(fpad, fpad, w2)


def _remap_body(idx_ref, gidx_ref):
    v = idx_ref[...]
    h = NPAD // 2
    kk = lax.broadcasted_iota(jnp.int32, (K, NPAD), 0)
    n = jnp.where(v < 0, N_NODES, v)
    gidx_ref[...] = jnp.where(n < h,
                              2 * (kk * h + n),
                              2 * (kk * h + n - h) + 1)


def _tc_remap(idx_t):
    return pl.pallas_call(
        _remap_body,
        out_shape=jax.ShapeDtypeStruct((K, NPAD), jnp.int32),
    )(idx_t)


def _sc_body(y_hbm, gidx_hbm, b_hbm, out_hbm,
             b_v, gidx_v, rows0, rows1, out0, out1,
             gsem0, gsem1, osem0, osem1):
    wid = lax.axis_index("s") * 2 + lax.axis_index("c")
    pltpu.sync_copy(b_hbm, b_v)
    pltpu.sync_copy(gidx_hbm.at[wid], gidx_v)
    wbase = wid * ROWS_PER_W
    himask = jnp.int32(-65536)

    rows = (rows0, rows1)
    gsems = (gsem0, gsem1)
    outs = (out0, out1)
    osems = (osem0, osem1)
    ostores = [None, None]

    def fire(c, par):
        return [pltpu.async_copy(y_hbm.at[gidx_v.at[k, c]], rows[par].at[k],
                                 gsems[par]) for k in range(K)]

    pending = fire(0, 0)
    for c in range(NCHUNK):
        par = c % 2
        nxt = fire(c + 1, 1 - par) if c + 1 < NCHUNK else []
        for cp in pending:
            cp.wait()
        pending = nxt
        buf = rows[par]
        outb = outs[par]
        if ostores[par] is not None:
            ostores[par].wait()

        def acc(i, carry2, buf=buf, outb=outb):
            for seg in range(4):
                s = pl.ds(seg * 16, 16)
                alo = b_v[s]
                ahi = b_v[pl.ds(64 + seg * 16, 16)]
                for k in range(K):
                    v = buf[k, i, s]
                    alo = alo + lax.bitcast_convert_type(v << 16, jnp.float32)
                    ahi = ahi + lax.bitcast_convert_type(v & himask,
                                                         jnp.float32)
                outb[i, s] = alo
                outb[i, pl.ds(64 + seg * 16, 16)] = ahi
            return carry2

        lax.fori_loop(0, B, acc, 0)
        ostores[par] = pltpu.async_copy(
            outb, out_hbm.at[pl.ds(wbase + c * B, B)], osems[par])
    ostores[0].wait()
    ostores[1].wait()


def _sc_gather_sum(yflat, gidx4, b):
    mesh = plsc.VectorSubcoreMesh(core_axis_name="c", subcore_axis_name="s")
    fn = pl.kernel(
        _sc_body,
        mesh=mesh,
        out_type=jax.ShapeDtypeStruct((NPAD, C_OUT), jnp.float32),
        compiler_params=pltpu.CompilerParams(use_tc_tiling_on_sc=False),
        scratch_types=[
            pltpu.VMEM((C_OUT,), jnp.float32),
            pltpu.VMEM((K, NCHUNK, B), jnp.int32),
            pltpu.VMEM((K, B, CW), jnp.int32),
            pltpu.VMEM((K, B, CW), jnp.int32),
            pltpu.VMEM((B, C_OUT), jnp.float32),
            pltpu.VMEM((B, C_OUT), jnp.float32),
            pltpu.SemaphoreType.DMA,
            pltpu.SemaphoreType.DMA,
            pltpu.SemaphoreType.DMA,
            pltpu.SemaphoreType.DMA,
        ],
    )
    return fn(yflat, gidx4, b)


def kernel(features, neigh_idx, W, b):
    N, C = features.shape
    fpad = jnp.zeros((NPAD, C), jnp.bfloat16).at[:N].set(
        features.astype(jnp.bfloat16))
    w2 = W.reshape(C_OUT, K, C_IN).transpose(2, 1, 0).reshape(C_IN, K * C_OUT)
    y = _tc_matmul(fpad, w2.astype(jnp.bfloat16))
    yflat = y.reshape(K * NPAD, CW)
    idx_pad = jnp.full((NPAD, K), -1, neigh_idx.dtype).at[:N].set(neigh_idx)
    idx_t = idx_pad.T.astype(jnp.int32)
    gidx = _tc_remap(idx_t)
    gidx4 = gidx.reshape(K, NW, NCHUNK, B).transpose(1, 0, 2, 3)
    out = _sc_gather_sum(yflat, gidx4, b)
    return out[:N]
